# baseline (speedup 1.0000x reference)
import jax
import jax.numpy as jnp
from jax.experimental import pallas as pl
from jax.experimental.pallas import tpu as pltpu


def _body(f_ref, v_ref, o_ref):
    x = f_ref[...]
    v = v_ref[...]
    l1 = jnp.sum(jnp.abs(x), axis=0, keepdims=True)
    speed = jnp.sum(jnp.abs(v), axis=0, keepdims=True)
    diff = l1 / jnp.maximum(l1, 1e-6)
    gate = jnp.exp(-diff) / (1.0 + 0.25 * speed)
    o_ref[...] = x * (1.0 - gate)


def kernel(coords, feats, vel_xy):
    del coords
    n, width = feats.shape
    ft = feats.T
    vt = vel_xy.T
    b = 100096
    grid = (pl.cdiv(n, b),)

    out_t = pl.pallas_call(
        _body,
        grid=grid,
        in_specs=[
            pl.BlockSpec((width, b), lambda i: (i * 0, i)),
            pl.BlockSpec((2, b), lambda i: (i * 0, i)),
        ],
        out_specs=pl.BlockSpec((width, b), lambda i: (i * 0, i)),
        out_shape=jax.ShapeDtypeStruct((width, n), jnp.float32),
        compiler_params=pltpu.CompilerParams(
            dimension_semantics=("arbitrary",),
        ),
    )(ft, vt)
    return out_t.T


# speedup vs baseline: 1.0159x; 1.0159x over previous
import jax
import jax.numpy as jnp
from jax.experimental import pallas as pl
from jax.experimental.pallas import tpu as pltpu


def _body(f_ref, v_ref, o_ref):
    x = f_ref[...]
    v = v_ref[...]
    l1 = jnp.sum(jnp.abs(x), axis=0, keepdims=True)
    speed = jnp.sum(jnp.abs(v), axis=0, keepdims=True)
    diff = l1 / jnp.maximum(l1, 1e-6)
    gate = jnp.exp(-diff) / (1.0 + 0.25 * speed)
    o_ref[...] = x * (1.0 - gate)


def kernel(coords, feats, vel_xy):
    del coords
    n, width = feats.shape
    ft = feats.T
    vt = vel_xy.T
    b = 73728
    grid = (pl.cdiv(n, b),)

    out_t = pl.pallas_call(
        _body,
        grid=grid,
        in_specs=[
            pl.BlockSpec((width, b), lambda i: (i * 0, i)),
            pl.BlockSpec((2, b), lambda i: (i * 0, i)),
        ],
        out_specs=pl.BlockSpec((width, b), lambda i: (i * 0, i)),
        out_shape=jax.ShapeDtypeStruct((width, n), jnp.float32),
        compiler_params=pltpu.CompilerParams(
            dimension_semantics=("arbitrary",),
        ),
    )(ft, vt)
    return out_t.T
